# baseline (speedup 1.0000x reference)
import functools

import jax
import jax.numpy as jnp
from jax.experimental import pallas as pl


def _proj_kernel(ns_ref, wq_ref, wk_ref, wv_ref, q_ref, k_ref, v_ref):
    ns = ns_ref[...]
    q_ref[...] = jax.lax.dot_general(
        ns, wq_ref[...], (((1,), (1,)), ((), ())),
        preferred_element_type=jnp.float32)
    k_ref[...] = jax.lax.dot_general(
        ns, wk_ref[...], (((1,), (1,)), ((), ())),
        preferred_element_type=jnp.float32)
    v_ref[...] = jax.lax.dot_general(
        ns, wv_ref[...], (((1,), (1,)), ((), ())),
        preferred_element_type=jnp.float32)


def _attn_kernel(q_ref, ns_ref, hid_ref, adj_ref, k_ref, v_ref,
                 w1_ref, b1_ref, w2_ref, b2_ref, step_ref, out_ref):
    q = q_ref[...]
    k = k_ref[...]
    v = v_ref[...]
    adj = adj_ref[...]

    scores_t = jax.lax.dot_general(
        k, q, (((1,), (1,)), ((), ())), preferred_element_type=jnp.float32)
    neg = jnp.float32(-1e30)
    masked = jnp.where(adj, scores_t, neg)
    m = jnp.max(masked, axis=0, keepdims=True)
    e = jnp.where(adj, jnp.exp(masked - m), 0.0)
    denom = jnp.sum(e, axis=0, keepdims=True)
    p = e / denom
    acc = jax.lax.dot_general(
        p, v, (((0,), (0,)), ((), ())), preferred_element_type=jnp.float32)

    nps = jnp.concatenate([ns_ref[...], acc], axis=1)
    h = jax.lax.dot_general(
        nps, w1_ref[...], (((1,), (1,)), ((), ())),
        preferred_element_type=jnp.float32) + b1_ref[...]
    h = jnp.maximum(h, 0.0)
    upd = jax.lax.dot_general(
        h, w2_ref[...], (((1,), (1,)), ((), ())),
        preferred_element_type=jnp.float32) + b2_ref[...]
    out_ref[...] = hid_ref[...] + step_ref[0, 0] * upd


@functools.partial(jax.jit, static_argnames=())
def kernel(input_states, hidden_states, adjacency_matrix, Wq, Wk, Wv,
           W1, b1, W2, b2, step_size):
    n, in_sz = input_states.shape
    hid_sz = hidden_states.shape[1]
    d = in_sz + hid_sz
    mlp_h = W1.shape[0]

    ns = jnp.concatenate([input_states, hidden_states], axis=1)

    q, k, v = pl.pallas_call(
        _proj_kernel,
        out_shape=[jax.ShapeDtypeStruct((n, d), jnp.float32)] * 3,
    )(ns, Wq, Wk, Wv)

    r = 256
    grid = (n // r,)
    out = pl.pallas_call(
        _attn_kernel,
        grid=grid,
        in_specs=[
            pl.BlockSpec((r, d), lambda i: (i, 0)),
            pl.BlockSpec((r, d), lambda i: (i, 0)),
            pl.BlockSpec((r, hid_sz), lambda i: (i, 0)),
            pl.BlockSpec((n, r), lambda i: (0, i)),
            pl.BlockSpec((n, d), lambda i: (0, 0)),
            pl.BlockSpec((n, d), lambda i: (0, 0)),
            pl.BlockSpec((mlp_h, 2 * d), lambda i: (0, 0)),
            pl.BlockSpec((1, mlp_h), lambda i: (0, 0)),
            pl.BlockSpec((hid_sz, mlp_h), lambda i: (0, 0)),
            pl.BlockSpec((1, hid_sz), lambda i: (0, 0)),
            pl.BlockSpec((1, 1), lambda i: (0, 0)),
        ],
        out_specs=pl.BlockSpec((r, hid_sz), lambda i: (i, 0)),
        out_shape=jax.ShapeDtypeStruct((n, hid_sz), jnp.float32),
    )(q, ns, hidden_states, adjacency_matrix, k, v,
      W1, b1.reshape(1, mlp_h), W2, b2.reshape(1, hid_sz),
      step_size.reshape(1, 1))
    return out


# speedup vs baseline: 1.1274x; 1.1274x over previous
import functools

import jax
import jax.numpy as jnp
from jax.experimental import pallas as pl


def _proj_kernel(ns_ref, wq_ref, wk_ref, wv_ref, q_ref, k_ref, v_ref):
    ns = ns_ref[...]
    q_ref[...] = jax.lax.dot_general(
        ns, wq_ref[...], (((1,), (1,)), ((), ())),
        preferred_element_type=jnp.float32)
    k_ref[...] = jax.lax.dot_general(
        ns, wk_ref[...], (((1,), (1,)), ((), ())),
        preferred_element_type=jnp.float32)
    v_ref[...] = jax.lax.dot_general(
        ns, wv_ref[...], (((1,), (1,)), ((), ())),
        preferred_element_type=jnp.float32)


def _attn_kernel(q_ref, ns_ref, hid_ref, adj_ref, k_ref, v_ref,
                 w1_ref, b1_ref, w2_ref, b2_ref, step_ref, out_ref):
    q = q_ref[...]
    k = k_ref[...]
    v = v_ref[...]
    adj = adj_ref[...]

    scores_t = jax.lax.dot_general(
        k, q, (((1,), (1,)), ((), ())), preferred_element_type=jnp.float32)
    neg = jnp.float32(-1e30)
    masked = jnp.where(adj, scores_t, neg)
    m = jnp.max(masked, axis=0, keepdims=True)
    e = jnp.exp2(masked - m)
    denom = jnp.sum(e, axis=0, keepdims=True)
    accn = jax.lax.dot_general(
        e, v, (((0,), (0,)), ((), ())), preferred_element_type=jnp.float32)
    acc = accn * (1.0 / denom).reshape(-1, 1)

    nps = jnp.concatenate([ns_ref[...], acc], axis=1)
    h = jax.lax.dot_general(
        nps, w1_ref[...], (((1,), (1,)), ((), ())),
        preferred_element_type=jnp.float32) + b1_ref[...]
    h = jnp.maximum(h, 0.0)
    upd = jax.lax.dot_general(
        h, w2_ref[...], (((1,), (1,)), ((), ())),
        preferred_element_type=jnp.float32) + b2_ref[...]
    out_ref[...] = hid_ref[...] + step_ref[0, 0] * upd


@functools.partial(jax.jit, static_argnames=())
def kernel(input_states, hidden_states, adjacency_matrix, Wq, Wk, Wv,
           W1, b1, W2, b2, step_size):
    n, in_sz = input_states.shape
    hid_sz = hidden_states.shape[1]
    d = in_sz + hid_sz
    mlp_h = W1.shape[0]

    ns = jnp.concatenate([input_states, hidden_states], axis=1)
    Wq = Wq * jnp.float32(1.4426950408889634)

    q, k, v = pl.pallas_call(
        _proj_kernel,
        out_shape=[jax.ShapeDtypeStruct((n, d), jnp.float32)] * 3,
    )(ns, Wq, Wk, Wv)

    r = 256
    grid = (n // r,)
    out = pl.pallas_call(
        _attn_kernel,
        grid=grid,
        in_specs=[
            pl.BlockSpec((r, d), lambda i: (i, 0)),
            pl.BlockSpec((r, d), lambda i: (i, 0)),
            pl.BlockSpec((r, hid_sz), lambda i: (i, 0)),
            pl.BlockSpec((n, r), lambda i: (0, i)),
            pl.BlockSpec((n, d), lambda i: (0, 0)),
            pl.BlockSpec((n, d), lambda i: (0, 0)),
            pl.BlockSpec((mlp_h, 2 * d), lambda i: (0, 0)),
            pl.BlockSpec((1, mlp_h), lambda i: (0, 0)),
            pl.BlockSpec((hid_sz, mlp_h), lambda i: (0, 0)),
            pl.BlockSpec((1, hid_sz), lambda i: (0, 0)),
            pl.BlockSpec((1, 1), lambda i: (0, 0)),
        ],
        out_specs=pl.BlockSpec((r, hid_sz), lambda i: (i, 0)),
        out_shape=jax.ShapeDtypeStruct((n, hid_sz), jnp.float32),
    )(q, ns, hidden_states, adjacency_matrix, k, v,
      W1, b1.reshape(1, mlp_h), W2, b2.reshape(1, hid_sz),
      step_size.reshape(1, 1))
    return out


# speedup vs baseline: 1.1411x; 1.0121x over previous
import functools

import jax
import jax.numpy as jnp
from jax.experimental import pallas as pl


def _proj_kernel(ns_ref, wq_ref, wk_ref, wv_ref, q_ref, k_ref, vb_ref):
    ns = ns_ref[...]
    q_ref[...] = jax.lax.dot_general(
        ns, wq_ref[...], (((1,), (1,)), ((), ())),
        preferred_element_type=jnp.float32)
    k_ref[...] = jax.lax.dot_general(
        ns, wk_ref[...], (((1,), (1,)), ((), ())),
        preferred_element_type=jnp.float32)
    v = jax.lax.dot_general(
        ns, wv_ref[...], (((1,), (1,)), ((), ())),
        preferred_element_type=jnp.float32)
    vb_ref[...] = v.astype(jnp.bfloat16)


def _attn_kernel(q_ref, ns_ref, hid_ref, adj_ref, k_ref, v_ref,
                 w1_ref, b1_ref, w2_ref, b2_ref, step_ref, out_ref):
    q = q_ref[...]
    k = k_ref[...]
    v = v_ref[...]
    adj = adj_ref[...]

    scores_t = jax.lax.dot_general(
        k, q, (((1,), (1,)), ((), ())), preferred_element_type=jnp.float32)
    neg = jnp.float32(-1e30)
    m = jnp.max(jnp.where(adj, scores_t, neg), axis=0, keepdims=True)
    e = jnp.where(adj, jnp.exp2(scores_t - m), 0.0).astype(jnp.bfloat16)
    denom = jnp.sum(e, axis=0, keepdims=True, dtype=jnp.float32)
    accn = jax.lax.dot_general(
        e, v, (((0,), (0,)), ((), ())), preferred_element_type=jnp.float32)
    acc = accn * (1.0 / denom).reshape(-1, 1)

    nps = jnp.concatenate([ns_ref[...], acc], axis=1)
    h = jax.lax.dot_general(
        nps, w1_ref[...], (((1,), (1,)), ((), ())),
        preferred_element_type=jnp.float32) + b1_ref[...]
    h = jnp.maximum(h, 0.0)
    upd = jax.lax.dot_general(
        h, w2_ref[...], (((1,), (1,)), ((), ())),
        preferred_element_type=jnp.float32) + b2_ref[...]
    out_ref[...] = hid_ref[...] + step_ref[0, 0] * upd


@functools.partial(jax.jit, static_argnames=())
def kernel(input_states, hidden_states, adjacency_matrix, Wq, Wk, Wv,
           W1, b1, W2, b2, step_size):
    n, in_sz = input_states.shape
    hid_sz = hidden_states.shape[1]
    d = in_sz + hid_sz
    mlp_h = W1.shape[0]

    ns = jnp.concatenate([input_states, hidden_states], axis=1)
    Wq = Wq * jnp.float32(1.4426950408889634)

    q, k, v = pl.pallas_call(
        _proj_kernel,
        out_shape=[jax.ShapeDtypeStruct((n, d), jnp.float32),
                   jax.ShapeDtypeStruct((n, d), jnp.float32),
                   jax.ShapeDtypeStruct((n, d), jnp.bfloat16)],
    )(ns, Wq, Wk, Wv)

    r = 256
    grid = (n // r,)
    out = pl.pallas_call(
        _attn_kernel,
        grid=grid,
        in_specs=[
            pl.BlockSpec((r, d), lambda i: (i, 0)),
            pl.BlockSpec((r, d), lambda i: (i, 0)),
            pl.BlockSpec((r, hid_sz), lambda i: (i, 0)),
            pl.BlockSpec((n, r), lambda i: (0, i)),
            pl.BlockSpec((n, d), lambda i: (0, 0)),
            pl.BlockSpec((n, d), lambda i: (0, 0)),
            pl.BlockSpec((mlp_h, 2 * d), lambda i: (0, 0)),
            pl.BlockSpec((1, mlp_h), lambda i: (0, 0)),
            pl.BlockSpec((hid_sz, mlp_h), lambda i: (0, 0)),
            pl.BlockSpec((1, hid_sz), lambda i: (0, 0)),
            pl.BlockSpec((1, 1), lambda i: (0, 0)),
        ],
        out_specs=pl.BlockSpec((r, hid_sz), lambda i: (i, 0)),
        out_shape=jax.ShapeDtypeStruct((n, hid_sz), jnp.float32),
    )(q, ns, hidden_states, adjacency_matrix, k, v,
      W1, b1.reshape(1, mlp_h), W2, b2.reshape(1, hid_sz),
      step_size.reshape(1, 1))
    return out
